# baseline (speedup 1.0000x reference)
import functools

import jax
import jax.numpy as jnp
from jax import lax
from jax.experimental import pallas as pl
from jax.experimental.pallas import tpu as pltpu
from jax.experimental.pallas import tpu_sc as plsc

_BATCH = 16384
_DIM = 32
_V = 1000000
_NW = 32
_CW = _BATCH // _NW
_ROWS = _V // 2

_mesh = plsc.VectorSubcoreMesh(core_axis_name="c", subcore_axis_name="s")


@functools.partial(
    pl.kernel,
    mesh=_mesh,
    out_type=jax.ShapeDtypeStruct((2 * _DIM * _BATCH,), jnp.float32),
    compiler_params=pltpu.CompilerParams(needs_layout_passes=False),
    scratch_types=[
        pltpu.VMEM((_CW,), jnp.int32),
        pltpu.VMEM((_CW,), jnp.int32),
        pltpu.VMEM((_CW, 128), jnp.float32),
        pltpu.VMEM((_DIM * _CW,), jnp.float32),
        pltpu.VMEM((_DIM * _CW,), jnp.float32),
        pltpu.SemaphoreType.DMA,
    ],
)
def _rotate_lookup(tab, idx, out, idx_c, row_v, rows_b, vre, vim, sem):
    wid = lax.axis_index("s") * 2 + lax.axis_index("c")
    c0 = wid * _CW
    pltpu.sync_copy(idx.at[pl.ds(c0, _CW)], idx_c)
    for g in range(_CW // 16):
        iv = idx_c[pl.ds(g * 16, 16)]
        row_v[pl.ds(g * 16, 16)] = lax.shift_right_logical(iv, 1)
    pltpu.async_copy(tab.at[row_v], rows_b, sem).wait()
    slot16 = lax.iota(jnp.int32, 16)

    def _extract(g, _):
        iv = idx_c[pl.ds(g * 16, 16)]
        lane0 = (iv & 1) * (2 * _DIM)
        rows = slot16 + g * 16
        for f in range(_DIM):
            vre[pl.ds(f * _CW + g * 16, 16)] = plsc.load_gather(
                rows_b, [rows, lane0 + f])
            vim[pl.ds(f * _CW + g * 16, 16)] = plsc.load_gather(
                rows_b, [rows, lane0 + _DIM + f])
        return ()

    lax.fori_loop(0, _CW // 16, _extract, ())
    for f in range(_DIM):
        pltpu.sync_copy(vre.at[pl.ds(f * _CW, _CW)],
                        out.at[pl.ds(f * _BATCH + c0, _CW)])
        pltpu.sync_copy(vim.at[pl.ds(f * _CW, _CW)],
                        out.at[pl.ds((_DIM + f) * _BATCH + c0, _CW)])


def kernel(entity_idx, ent_re, ent_im):
    idx = entity_idx.astype(jnp.int32)
    tab = jnp.concatenate([ent_re.reshape(_ROWS, 2, _DIM),
                           ent_im.reshape(_ROWS, 2, _DIM)],
                          axis=2).reshape(_ROWS, 128)
    out = _rotate_lookup(tab, idx)
    return out.reshape(2 * _DIM, _BATCH).T


# speedup vs baseline: 1.8609x; 1.8609x over previous
import functools

import jax
import jax.numpy as jnp
from jax import lax
from jax.experimental import pallas as pl
from jax.experimental.pallas import tpu as pltpu
from jax.experimental.pallas import tpu_sc as plsc

_BATCH = 16384
_DIM = 32
_V = 1000000
_NW = 32
_CW = _BATCH // _NW
_H = _CW // 2
_ROWS = _V * _DIM // 128

_mesh = plsc.VectorSubcoreMesh(core_axis_name="c", subcore_axis_name="s")


@functools.partial(
    pl.kernel,
    mesh=_mesh,
    out_type=jax.ShapeDtypeStruct((2 * _DIM * _BATCH,), jnp.float32),
    compiler_params=pltpu.CompilerParams(needs_layout_passes=False),
    scratch_types=[
        pltpu.VMEM((_CW,), jnp.int32),
        pltpu.VMEM((_H,), jnp.int32),
        pltpu.VMEM((_H, 128), jnp.float32),
        pltpu.VMEM((_H, 128), jnp.float32),
        pltpu.VMEM((_DIM * _H,), jnp.float32),
        pltpu.VMEM((_DIM * _H,), jnp.float32),
        pltpu.SemaphoreType.DMA,
    ],
)
def _rotate_lookup(re2d, im2d, idx, out, idx_c, row_v, rre, rim, vre, vim,
                   sem):
    wid = lax.axis_index("s") * 2 + lax.axis_index("c")
    c0 = wid * _CW
    pltpu.sync_copy(idx.at[pl.ds(c0, _CW)], idx_c)
    slot16 = lax.iota(jnp.int32, 16)
    for h in range(2):
        hb = h * _H
        for g in range(_H // 16):
            iv = idx_c[pl.ds(hb + g * 16, 16)]
            row_v[pl.ds(g * 16, 16)] = lax.shift_right_logical(iv, 2)
        c1 = pltpu.async_copy(re2d.at[row_v], rre, sem)
        c2 = pltpu.async_copy(im2d.at[row_v], rim, sem)
        c1.wait()
        c2.wait()

        def _extract(g, _):
            iv = idx_c[pl.ds(hb + g * 16, 16)]
            lane0 = (iv & 3) * _DIM
            rows = slot16 + g * 16
            for f in range(_DIM):
                lanes = lane0 + f
                vre[pl.ds(f * _H + g * 16, 16)] = plsc.load_gather(
                    rre, [rows, lanes])
                vim[pl.ds(f * _H + g * 16, 16)] = plsc.load_gather(
                    rim, [rows, lanes])
            return ()

        lax.fori_loop(0, _H // 16, _extract, ())
        for f in range(_DIM):
            pltpu.sync_copy(vre.at[pl.ds(f * _H, _H)],
                            out.at[pl.ds(f * _BATCH + c0 + hb, _H)])
            pltpu.sync_copy(vim.at[pl.ds(f * _H, _H)],
                            out.at[pl.ds((_DIM + f) * _BATCH + c0 + hb, _H)])


def kernel(entity_idx, ent_re, ent_im):
    idx = entity_idx.astype(jnp.int32)
    re2d = ent_re.reshape(_ROWS, 128)
    im2d = ent_im.reshape(_ROWS, 128)
    out = _rotate_lookup(re2d, im2d, idx)
    return out.reshape(2 * _DIM, _BATCH).T
